# Initial kernel scaffold; baseline (speedup 1.0000x reference)
import jax
import jax.numpy as jnp
from jax.experimental import pallas as pl


def kernel(x, edge_index, W1, b1, W2, b2, W3, b3):
    raise NotImplementedError("write your pallas kernel here")


# speedup vs baseline: 3.6893x; 3.6893x over previous
import functools

import jax
import jax.numpy as jnp
from jax import lax
from jax.experimental import pallas as pl
from jax.experimental.pallas import tpu as pltpu
from jax.experimental.pallas import tpu_sc as plsc

N = 10000
NP = 10400
E = 160000
EP = 163840
IN_DIM = 256
HID_DIM = 512
CW = 64
NC1 = IN_DIM // CW
NC2 = HID_DIM // CW
NC, NS = 2, 16
BATCH = 128
NBATCH = EP // (NS * BATCH)
ROWS_PT = 648
BN = 400
NBLK = N // BN
NPBLK = NP // BN


def _zero_vmem_2d(ref, nrows, ncols, value=0.0):
    v16 = jnp.full((16,), value, jnp.float32)
    @pl.loop(0, nrows)
    def _(r):
        for i in range(ncols // 16):
            ref[r, pl.ds(i * 16, 16)] = v16


def _idx_shift(sidx, delta):
    @pl.loop(0, NBATCH)
    def _(b):
        for i in range(BATCH // 16):
            sidx[b, pl.ds(i * 16, 16)] = sidx[b, pl.ds(i * 16, 16)] + delta


def _tile_slab_copy(src_at, dst_at, s):
    off = s * ROWS_PT
    for j in range(5):
        pltpu.sync_copy(src_at(off + j * 128, 128), dst_at(off + j * 128, 128))
    pltpu.sync_copy(src_at(off + 640, 8), dst_at(off + 640, 8))
    @pl.when(s == NS - 1)
    def _():
        pltpu.sync_copy(src_at(NP - 32, 32), dst_at(NP - 32, 32))


def _agg_pass(table_hbm, out_hbm, sidx, didx, rows, Zsp, sem, s, chunk):
    _zero_vmem_2d(rows, BATCH, CW)
    _tile_slab_copy(lambda o, n: rows.at[pl.ds(0, n)],
                    lambda o, n: Zsp.at[pl.ds(o, n)], s)
    plsc.subcore_barrier()

    @pl.loop(0, NBATCH)
    def _(b):
        pltpu.async_copy(table_hbm.at[sidx.at[b]], rows, sem).wait()
        pltpu.sync_copy(rows, Zsp.at[didx.at[b]], add=True)

    plsc.subcore_barrier()
    _tile_slab_copy(lambda o, n: Zsp.at[pl.ds(o, n)],
                    lambda o, n: out_hbm.at[pl.ds(chunk * NP + o, n)], s)


def _sc_agg1_body(xc_hbm, src_hbm, dst_hbm, z1_hbm, degs_hbm,
                  sidx, didx, rows, ones16, z16, Zsp, degsp, sem):
    c = lax.axis_index("c")
    s = lax.axis_index("s")

    pltpu.sync_copy(src_hbm.at[pl.ds(s * NBATCH, NBATCH)], sidx)
    pltpu.sync_copy(dst_hbm.at[pl.ds(s * NBATCH, NBATCH)], didx)

    _zero_vmem_2d(z16, BATCH, 16)
    _zero_vmem_2d(ones16, BATCH, 16, value=1.0)
    _tile_slab_copy(lambda o, n: z16.at[pl.ds(0, n)],
                    lambda o, n: degsp.at[pl.ds(o, n)], s)
    plsc.subcore_barrier()

    @pl.loop(0, NBATCH)
    def _(b):
        @pl.when(c == 0)
        def _():
            pltpu.sync_copy(ones16, degsp.at[didx.at[b]], add=True)
        @pl.when(c == 1)
        def _():
            pltpu.sync_copy(ones16, degsp.at[sidx.at[b]], add=True)

    plsc.subcore_barrier()
    _tile_slab_copy(lambda o, n: degsp.at[pl.ds(o, n)],
                    lambda o, n: degs_hbm.at[pl.ds(c * NP + o, n)], s)

    _idx_shift(sidx, (2 * c) * NP)
    _agg_pass(xc_hbm, z1_hbm, sidx, didx, rows, Zsp, sem, s, 2 * c)
    _idx_shift(sidx, NP)
    _agg_pass(xc_hbm, z1_hbm, sidx, didx, rows, Zsp, sem, s, 2 * c + 1)


def _sc_agg2_body(t1_hbm, src_hbm, dst_hbm, z2_hbm,
                  sidx, didx, rows, Zsp, sem):
    c = lax.axis_index("c")
    s = lax.axis_index("s")

    pltpu.sync_copy(src_hbm.at[pl.ds(s * NBATCH, NBATCH)], sidx)
    pltpu.sync_copy(dst_hbm.at[pl.ds(s * NBATCH, NBATCH)], didx)

    _idx_shift(sidx, (4 * c) * NP)
    _agg_pass(t1_hbm, z2_hbm, sidx, didx, rows, Zsp, sem, s, 4 * c)
    for p in range(1, 4):
        _idx_shift(sidx, NP)
        _agg_pass(t1_hbm, z2_hbm, sidx, didx, rows, Zsp, sem, s, 4 * c + p)


@functools.lru_cache(maxsize=1)
def _get_sc_kernels():
    mesh = plsc.VectorSubcoreMesh(
        core_axis_name="c", subcore_axis_name="s",
        num_cores=NC, num_subcores=NS)
    agg1 = functools.partial(
        pl.kernel,
        out_type=(
            jax.ShapeDtypeStruct((NC1 * NP, CW), jnp.float32),
            jax.ShapeDtypeStruct((2 * NP, 16), jnp.float32),
        ),
        mesh=mesh,
        compiler_params=pltpu.CompilerParams(use_tc_tiling_on_sc=False),
        scratch_types=(
            pltpu.VMEM((NBATCH, BATCH), jnp.int32),
            pltpu.VMEM((NBATCH, BATCH), jnp.int32),
            pltpu.VMEM((BATCH, CW), jnp.float32),
            pltpu.VMEM((BATCH, 16), jnp.float32),
            pltpu.VMEM((BATCH, 16), jnp.float32),
            pltpu.VMEM_SHARED((NP, CW), jnp.float32),
            pltpu.VMEM_SHARED((NP, 16), jnp.float32),
            pltpu.SemaphoreType.DMA,
        ),
    )(_sc_agg1_body)
    agg2 = functools.partial(
        pl.kernel,
        out_type=jax.ShapeDtypeStruct((NC2 * NP, CW), jnp.float32),
        mesh=mesh,
        compiler_params=pltpu.CompilerParams(use_tc_tiling_on_sc=False),
        scratch_types=(
            pltpu.VMEM((NBATCH, BATCH), jnp.int32),
            pltpu.VMEM((NBATCH, BATCH), jnp.int32),
            pltpu.VMEM((BATCH, CW), jnp.float32),
            pltpu.VMEM_SHARED((NP, CW), jnp.float32),
            pltpu.SemaphoreType.DMA,
        ),
    )(_sc_agg2_body)
    return agg1, agg2


def _tc1_body(z0_ref, z1_ref, z2_ref, z3_ref, w1t_ref, b1_ref, deg_ref,
              out_ref):
    w = w1t_ref[...]
    h = jnp.dot(z0_ref[...], w[0:64, :], preferred_element_type=jnp.float32)
    h += jnp.dot(z1_ref[...], w[64:128, :], preferred_element_type=jnp.float32)
    h += jnp.dot(z2_ref[...], w[128:192, :], preferred_element_type=jnp.float32)
    h += jnp.dot(z3_ref[...], w[192:256, :], preferred_element_type=jnp.float32)
    h += deg_ref[:, 0:1] * b1_ref[...]
    t = jnp.tanh(h)
    out_ref[0] = t[:, 0:CW]
    out_ref[1] = t[:, CW:2 * CW]


def _tc2_body(z0_ref, z1_ref, z2_ref, z3_ref, z4_ref, z5_ref, z6_ref, z7_ref,
              w2t_ref, b2_ref, degin_ref, degout_ref, w3_ref, b3_ref,
              out_ref):
    i = pl.program_id(0)
    w = w2t_ref[...]
    zrefs = [z0_ref, z1_ref, z2_ref, z3_ref, z4_ref, z5_ref, z6_ref, z7_ref]
    h = jnp.dot(zrefs[0][...], w[0:CW, :], preferred_element_type=jnp.float32)
    for k in range(1, NC2):
        h += jnp.dot(zrefs[k][...], w[k * CW:(k + 1) * CW, :],
                     preferred_element_type=jnp.float32)
    h += degin_ref[:, 0:1] * b2_ref[...]
    t2 = jnp.tanh(h)
    v3 = jnp.sum(w3_ref[...], axis=0, keepdims=True)
    srow = jnp.sum(t2 * v3, axis=1)
    part = jnp.sum(degout_ref[:, 0] * srow)
    part += jnp.where(i == 0, float(E) * jnp.sum(b3_ref[...]), 0.0)

    @pl.when(i == 0)
    def _():
        out_ref[...] = jnp.zeros((8, 128), jnp.float32)
    r = lax.broadcasted_iota(jnp.int32, (8, 128), 0)
    col = lax.broadcasted_iota(jnp.int32, (8, 128), 1)
    mask = jnp.logical_and(r == 0, col == 0).astype(jnp.float32)
    out_ref[...] += part * mask


def kernel(x, edge_index, W1, b1, W2, b2, W3, b3):
    src = edge_index[0].astype(jnp.int32)
    dst = edge_index[1].astype(jnp.int32)
    padi = jnp.full((EP - E,), N, jnp.int32)
    src2 = jnp.concatenate([src, padi]).reshape(EP // BATCH, BATCH)
    dst2 = jnp.concatenate([dst, padi]).reshape(EP // BATCH, BATCH)

    xc = x.astype(jnp.float32).reshape(N, NC1, CW).transpose(1, 0, 2)
    xc = jnp.pad(xc, ((0, 0), (0, NP - N), (0, 0))).reshape(NC1 * NP, CW)

    sc_agg1, sc_agg2 = _get_sc_kernels()
    z1c, degs = sc_agg1(xc, src2, dst2)

    w1t = W1.T.astype(jnp.float32)
    b1r = b1.astype(jnp.float32).reshape(1, HID_DIM)
    t1c = pl.pallas_call(
        _tc1_body,
        grid=(NPBLK, NC2 // 2),
        in_specs=[
            pl.BlockSpec((BN, CW), lambda i, j: (i, 0)),
            pl.BlockSpec((BN, CW), lambda i, j: (NPBLK + i, 0)),
            pl.BlockSpec((BN, CW), lambda i, j: (2 * NPBLK + i, 0)),
            pl.BlockSpec((BN, CW), lambda i, j: (3 * NPBLK + i, 0)),
            pl.BlockSpec((IN_DIM, 128), lambda i, j: (0, j)),
            pl.BlockSpec((1, 128), lambda i, j: (0, j)),
            pl.BlockSpec((BN, 16), lambda i, j: (i, 0)),
        ],
        out_specs=pl.BlockSpec((2, BN, CW), lambda i, j: (j, i, 0)),
        out_shape=jax.ShapeDtypeStruct((NC2, NP, CW), jnp.float32),
    )(z1c, z1c, z1c, z1c, w1t, b1r, degs)
    t1c = t1c.reshape(NC2 * NP, CW)

    z2c = sc_agg2(t1c, src2, dst2)

    w2t = W2.T.astype(jnp.float32)
    b2r = b2.astype(jnp.float32).reshape(1, HID_DIM)
    b3r = b3.astype(jnp.float32).reshape(1, HID_DIM)
    zspecs = [
        pl.BlockSpec((BN, CW), (lambda k: (lambda i, _k=k: (_k * NPBLK + i, 0)))(k))
        for k in range(NC2)
    ]
    o8 = pl.pallas_call(
        _tc2_body,
        grid=(NBLK,),
        in_specs=zspecs + [
            pl.BlockSpec((HID_DIM, HID_DIM), lambda i: (0, 0)),
            pl.BlockSpec((1, HID_DIM), lambda i: (0, 0)),
            pl.BlockSpec((BN, 16), lambda i: (i, 0)),
            pl.BlockSpec((BN, 16), lambda i: (NPBLK + i, 0)),
            pl.BlockSpec((HID_DIM, HID_DIM), lambda i: (0, 0)),
            pl.BlockSpec((1, HID_DIM), lambda i: (0, 0)),
        ],
        out_specs=pl.BlockSpec((8, 128), lambda i: (0, 0)),
        out_shape=jax.ShapeDtypeStruct((8, 128), jnp.float32),
    )(*([z2c] * NC2), w2t, b2r, degs, degs, W3.astype(jnp.float32), b3r)

    return o8[0:1, 0:1]
